# Initial kernel scaffold; baseline (speedup 1.0000x reference)
import jax
import jax.numpy as jnp
from jax.experimental import pallas as pl


def kernel(keypoints, points_xyz, point_feats, spatial_features, w1, b1, w2, b2, w_fuse):
    raise NotImplementedError("write your pallas kernel here")


# speedup vs baseline: 4.9668x; 4.9668x over previous
import functools

import numpy as np
import jax
import jax.numpy as jnp
from jax import lax
from jax.experimental import pallas as pl
from jax.experimental.pallas import tpu as pltpu
from jax.experimental.pallas import tpu_sc as plsc

_F32 = jnp.float32
_R2 = np.float32(0.8 ** 2)
_BIG = np.float32(1e30)
_NS = 16
_TK = 256


def _select_body(kp_ref, pts_ref, nidx_ref, nmask_ref, bidx_ref, bw_ref,
                 d2_ref, *, n_pts, bev_h, bev_w):
    b = pl.program_id(0)
    kp = kp_ref[0]

    d2 = None
    for d in range(3):
        diff = kp[:, d:d + 1] - pts_ref[0, d:d + 1, :]
        sq = diff * diff
        d2 = sq if d2 is None else d2 + sq
    d2_ref[...] = jnp.where(d2 <= _R2, d2, _BIG)

    iota = lax.broadcasted_iota(jnp.int32, (_TK, n_pts), 1)
    base = b * n_pts
    idx_cols = []
    msk_cols = []
    for _ in range(_NS):
        d2m = d2_ref[...]
        m = jnp.min(d2m, axis=1, keepdims=True)
        am = jnp.min(jnp.where(d2m <= m, iota, n_pts), axis=1,
                     keepdims=True)
        idx_cols.append(am + base)
        msk_cols.append((m < _BIG).astype(_F32))
        d2_ref[...] = jnp.where(iota == am, _BIG, d2m)
    nidx_ref[0] = jnp.concatenate(idx_cols, axis=1)
    nmask_ref[0] = jnp.concatenate(msk_cols, axis=1)

    x = kp[:, 0:1] / np.float32(0.05) / np.float32(8.0)
    y = (kp[:, 1:2] - np.float32(-40.0)) / np.float32(0.05) / np.float32(8.0)
    x0 = jnp.floor(x)
    y0 = jnp.floor(y)
    x0i = x0.astype(jnp.int32)
    y0i = y0.astype(jnp.int32)
    x1i = x0i + 1
    y1i = y0i + 1
    x0c = jnp.clip(x0i, 0, bev_w - 1)
    x1c = jnp.clip(x1i, 0, bev_w - 1)
    y0c = jnp.clip(y0i, 0, bev_h - 1)
    y1c = jnp.clip(y1i, 0, bev_h - 1)
    x1f = x1i.astype(_F32)
    y1f = y1i.astype(_F32)
    x0f = x0i.astype(_F32)
    y0f = y0i.astype(_F32)
    wa = (x1f - x) * (y1f - y)
    wb = (x1f - x) * (y - y0f)
    wc = (x - x0f) * (y1f - y)
    wd = (x - x0f) * (y - y0f)
    bev_base = b * (bev_h * bev_w)
    ia = bev_base + y0c * bev_w + x0c
    ib = bev_base + y1c * bev_w + x0c
    ic = bev_base + y0c * bev_w + x1c
    idd = bev_base + y1c * bev_w + x1c
    bidx_ref[0] = jnp.concatenate([ia, ib, ic, idd], axis=1)
    bw_ref[0] = jnp.concatenate([wa, wb, wc, wd], axis=1)


def _select(keypoints, pts_t, n_pts, bev_h, bev_w):
    b_sz, k_sz, _ = keypoints.shape
    grid = (b_sz, k_sz // _TK)
    body = functools.partial(_select_body, n_pts=n_pts, bev_h=bev_h,
                             bev_w=bev_w)
    return pl.pallas_call(
        body,
        grid=grid,
        in_specs=[
            pl.BlockSpec((1, _TK, 3), lambda b, k: (b, k, 0)),
            pl.BlockSpec((1, 3, n_pts), lambda b, k: (b, 0, 0)),
        ],
        out_specs=[
            pl.BlockSpec((1, _TK, _NS), lambda b, k: (b, k, 0)),
            pl.BlockSpec((1, _TK, _NS), lambda b, k: (b, k, 0)),
            pl.BlockSpec((1, _TK, 4), lambda b, k: (b, k, 0)),
            pl.BlockSpec((1, _TK, 4), lambda b, k: (b, k, 0)),
        ],
        out_shape=[
            jax.ShapeDtypeStruct((b_sz, k_sz, _NS), jnp.int32),
            jax.ShapeDtypeStruct((b_sz, k_sz, _NS), _F32),
            jax.ShapeDtypeStruct((b_sz, k_sz, 4), jnp.int32),
            jax.ShapeDtypeStruct((b_sz, k_sz, 4), _F32),
        ],
        scratch_shapes=[pltpu.VMEM((_TK, n_pts), _F32)],
    )(keypoints, pts_t)


def _sc_gather(pts_table, bev_table, nidx_flat, bidx_flat):
    info = plsc.get_sparse_core_info()
    nc, ns = info.num_cores, info.num_subcores
    nw = nc * ns
    rn = nidx_flat.shape[0] // nw
    rb = bidx_flat.shape[0] // nw
    ch = 128
    c_pts = pts_table.shape[1]
    c_bev = bev_table.shape[1]
    mesh = plsc.VectorSubcoreMesh(core_axis_name="c", subcore_axis_name="s")

    @functools.partial(
        pl.kernel, mesh=mesh,
        out_type=(
            jax.ShapeDtypeStruct((nidx_flat.shape[0], c_pts), _F32),
            jax.ShapeDtypeStruct((bidx_flat.shape[0], c_bev), _F32),
        ),
        scratch_types=[
            pltpu.VMEM((ch,), jnp.int32),
            pltpu.VMEM((ch, c_pts), _F32),
            pltpu.VMEM((ch, c_bev), _F32),
            pltpu.SemaphoreType.DMA,
        ],
    )
    def k(pts_hbm, bev_hbm, nidx_hbm, bidx_hbm, nrows_out, brows_out,
          idx_v, nrow_v, brow_v, sem):
        wid = lax.axis_index("s") * nc + lax.axis_index("c")
        nbase = wid * rn
        for j in range(rn // ch):
            o = nbase + j * ch
            pltpu.sync_copy(nidx_hbm.at[pl.ds(o, ch)], idx_v)
            pltpu.async_copy(pts_hbm.at[idx_v], nrow_v, sem).wait()
            pltpu.sync_copy(nrow_v, nrows_out.at[pl.ds(o, ch)])
        bbase = wid * rb
        for j in range(rb // ch):
            o = bbase + j * ch
            pltpu.sync_copy(bidx_hbm.at[pl.ds(o, ch)], idx_v)
            pltpu.async_copy(bev_hbm.at[idx_v], brow_v, sem).wait()
            pltpu.sync_copy(brow_v, brows_out.at[pl.ds(o, ch)])

    return k(pts_table, bev_table, nidx_flat, bidx_flat)


def _fuse_body(kp_ref, nbr_ref, bev_ref, nmask_ref, bw_ref,
               w1_ref, b1_ref, w2_ref, b2_ref, wf_ref, out_ref):
    kp = kp_ref[0]
    w1 = w1_ref[...]
    g = nbr_ref[0]

    kpw = None
    for j in range(3):
        t = kp[:, j:j + 1] * w1[j][None, :]
        kpw = t if kpw is None else kpw + t
    h1 = None
    for j in range(4):
        t = g[:, :, j:j + 1] * w1[j][None, None, :]
        h1 = t if h1 is None else h1 + t
    h1 = jnp.maximum(h1 - kpw[:, None, :] + b1_ref[0][None, None, :], 0.0)

    w2 = w2_ref[...]
    h2 = None
    for j in range(_NS):
        t = h1[:, :, j:j + 1] * w2[j][None, None, :]
        h2 = t if h2 is None else h2 + t
    h2 = jnp.maximum(h2 + b2_ref[0][None, None, :], 0.0)
    h2 = h2 * nmask_ref[0][:, :, None]
    pooled = jnp.max(h2, axis=1)

    bev = bev_ref[0]
    w = bw_ref[0]
    feats = bev[:, 0, :] * w[:, 0:1]
    feats = feats + bev[:, 1, :] * w[:, 1:2]
    feats = feats + bev[:, 2, :] * w[:, 2:3]
    feats = feats + bev[:, 3, :] * w[:, 3:4]

    wf = wf_ref[...]
    c_bev = feats.shape[1]
    out = jnp.dot(feats, wf[:c_bev], preferred_element_type=_F32)
    out = out + jnp.dot(pooled, wf[c_bev:], preferred_element_type=_F32)
    out_ref[0] = jnp.maximum(out, 0.0)


def _fuse(keypoints, nbr4, bev4, nmask, bw, w1, b1, w2, b2, w_fuse):
    b_sz, k_sz, _ = keypoints.shape
    c_bev = bev4.shape[-1]
    c_pts = nbr4.shape[-1]
    c_out = w_fuse.shape[1]
    grid = (b_sz, k_sz // _TK)
    return pl.pallas_call(
        _fuse_body,
        grid=grid,
        in_specs=[
            pl.BlockSpec((1, _TK, 3), lambda b, k: (b, k, 0)),
            pl.BlockSpec((1, _TK, _NS, c_pts), lambda b, k: (b, k, 0, 0)),
            pl.BlockSpec((1, _TK, 4, c_bev), lambda b, k: (b, k, 0, 0)),
            pl.BlockSpec((1, _TK, _NS), lambda b, k: (b, k, 0)),
            pl.BlockSpec((1, _TK, 4), lambda b, k: (b, k, 0)),
            pl.BlockSpec((4, _NS), lambda b, k: (0, 0)),
            pl.BlockSpec((1, _NS), lambda b, k: (0, 0)),
            pl.BlockSpec((_NS, _NS), lambda b, k: (0, 0)),
            pl.BlockSpec((1, _NS), lambda b, k: (0, 0)),
            pl.BlockSpec(w_fuse.shape, lambda b, k: (0, 0)),
        ],
        out_specs=pl.BlockSpec((1, _TK, c_out), lambda b, k: (b, k, 0)),
        out_shape=jax.ShapeDtypeStruct((b_sz, k_sz, c_out), _F32),
    )(keypoints, nbr4, bev4, nmask, bw, w1, b1, w2, b2, w_fuse)


def kernel(keypoints, points_xyz, point_feats, spatial_features,
           w1, b1, w2, b2, w_fuse):
    b_sz, k_sz, _ = keypoints.shape
    n_pts = points_xyz.shape[1]
    c_bev, bev_h, bev_w = spatial_features.shape[1:]

    pts_t = jnp.transpose(points_xyz, (0, 2, 1))
    nidx, nmask, bidx, bw = _select(keypoints, pts_t, n_pts, bev_h, bev_w)

    pts_table = jnp.concatenate(
        [points_xyz, point_feats,
         jnp.zeros((b_sz, n_pts, 124), _F32)], axis=-1)
    pts_table = pts_table.reshape(b_sz * n_pts, 128)
    bev_table = jnp.transpose(spatial_features, (0, 2, 3, 1))
    bev_table = bev_table.reshape(b_sz * bev_h * bev_w, c_bev)

    nbr_rows, bev_rows = _sc_gather(pts_table, bev_table,
                                    nidx.reshape(-1), bidx.reshape(-1))
    nbr4 = nbr_rows.reshape(b_sz, k_sz, _NS, 128)
    bev4 = bev_rows.reshape(b_sz, k_sz, 4, c_bev)

    return _fuse(keypoints, nbr4, bev4, nmask, bw,
                 w1, b1.reshape(1, _NS), w2, b2.reshape(1, _NS), w_fuse)
